# baseline (speedup 1.0000x reference)
import jax
import jax.numpy as jnp
from jax import lax
from jax.experimental import pallas as pl
from jax.experimental.pallas import tpu as pltpu
from jax.experimental.pallas import tpu_sc as plsc

B = 4096
HIST = 20
D = 64
G = 3 * D
V = 1000000

BB = 256
NB = B // BB

NC, NS = 2, 16
NW = NC * NS
GN = B * HIST
G_PER_W = GN // NW
G_CH = G_PER_W // 4
S_PER_W = B // NW


def _sc_gather_body(item_t, user_t, u_idx, i_idx, g_u, g_i,
                    idx_v, rows_v, sem):
    wid = lax.axis_index("s") * NC + lax.axis_index("c")
    for table, idx_hbm, out_hbm in ((item_t, u_idx, g_u), (user_t, i_idx, g_i)):
        for c in range(G_PER_W // G_CH):
            base = wid * G_PER_W + c * G_CH
            pltpu.sync_copy(idx_hbm.at[pl.ds(base, G_CH)], idx_v)
            pltpu.async_copy(table.at[idx_v], rows_v, sem).wait()
            pltpu.sync_copy(rows_v, out_hbm.at[pl.ds(base, G_CH)])


PV = V // 2


import functools as _ft


@_ft.cache
def _make_sc_gather():
  return pl.kernel(
    _sc_gather_body,
    out_type=[jax.ShapeDtypeStruct((GN, 2 * D), jnp.float32)] * 2,
    mesh=plsc.VectorSubcoreMesh(core_axis_name="c", subcore_axis_name="s", num_cores=NC, num_subcores=NS),
    compiler_params=pltpu.CompilerParams(use_tc_tiling_on_sc=False),
    scratch_types=[
        pltpu.VMEM((G_CH,), jnp.int32),
        pltpu.VMEM((G_CH, 2 * D), jnp.float32),
        pltpu.SemaphoreType.DMA,
    ],
  )


def _tc_body(w_e, wd_c, bih_c, w_hh, bhh_c, gu, gi_, mu, mi, du, di,
             ue_out, ie_out,
             x_ref, hu_ref, hi_ref, ulog_ref):
    we = w_e[...]
    whh = w_hh[...]
    b_i = bih_c[...]
    b_h = bhh_c[...]
    wdv = wd_c[...]

    nt = (((1,), (1,)), ((), ()))
    for e, g, mk, dl in ((0, gu, mu, du), (1, gi_, mi, di)):
        e128 = g[...]
        lo = e128[:, :D]
        hi = e128[:, D:]
        m = mk[...]
        emb = lo + m * (hi - lo)
        giT = lax.dot_general(we, emb, nt, preferred_element_type=jnp.float32)
        giT = giT + wdv * dl[...].reshape(1, HIST * BB) + b_i
        x_ref[e] = giT

    nn = (((1,), (0,)), ((), ()))

    def step(t, HT):
        xu = x_ref[0, :, pl.ds(t * BB, BB)]
        xi = x_ref[1, :, pl.ds(t * BB, BB)]
        xt = jnp.concatenate([xu, xi], axis=1)
        gh = lax.dot_general(whh, HT, nn, preferred_element_type=jnp.float32) + b_h
        r = jax.nn.sigmoid(xt[:D] + gh[:D])
        z = jax.nn.sigmoid(xt[D:2 * D] + gh[D:2 * D])
        n = jnp.tanh(xt[2 * D:] + r * gh[2 * D:])
        HTn = (1.0 - z) * n + z * HT
        hu_ref[t] = HTn[:, :BB]
        hi_ref[t] = HTn[:, BB:]
        return HTn

    lax.fori_loop(0, HIST, step, jnp.zeros((D, 2 * BB), jnp.float32))

    def att_h(h, i_acc):
        u_h = hu_ref[h]
        ths = []
        for k in range(HIST):
            ths.append(jnp.tanh(jnp.sum(u_h * hi_ref[k], axis=0)))
        th = jnp.stack(ths, axis=0)
        ulog_ref[h] = jnp.mean(th, axis=0)
        return i_acc + th

    i_sum = lax.fori_loop(0, HIST, att_h, jnp.zeros((HIST, BB), jnp.float32))

    def soft(x):
        m = jnp.max(x, axis=0, keepdims=True)
        e = jnp.exp(x - m)
        return e / jnp.sum(e, axis=0, keepdims=True)

    u_att = soft(ulog_ref[...])
    i_att = soft(i_sum / HIST)
    ueT = jnp.zeros((D, BB), jnp.float32)
    ieT = jnp.zeros((D, BB), jnp.float32)
    for h in range(HIST):
        ueT = ueT + u_att[h][None, :] * hu_ref[h]
        ieT = ieT + i_att[h][None, :] * hi_ref[h]
    ue_out[...] = ueT.T
    ie_out[...] = ieT.T


def _tc_main(w_e, wd_c, bih_c, w_hh, bhh_c, g_u, g_i, m_u, m_i, du, di):
    full = lambda s: pl.BlockSpec(s, lambda i: (0,) * len(s))
    return pl.pallas_call(
        _tc_body,
        grid=(NB,),
        in_specs=[
            full((G, D)), full((G, 1)), full((G, 1)), full((G, D)), full((G, 1)),
            pl.BlockSpec((HIST * BB, 2 * D), lambda i: (i, 0)),
            pl.BlockSpec((HIST * BB, 2 * D), lambda i: (i, 0)),
            pl.BlockSpec((HIST * BB, D), lambda i: (i, 0)),
            pl.BlockSpec((HIST * BB, D), lambda i: (i, 0)),
            pl.BlockSpec((1, 1, HIST * BB), lambda i: (i, 0, 0)),
            pl.BlockSpec((1, 1, HIST * BB), lambda i: (i, 0, 0)),
        ],
        out_specs=[pl.BlockSpec((BB, D), lambda i: (i, 0))] * 2,
        out_shape=[jax.ShapeDtypeStruct((B, D), jnp.float32)] * 2,
        scratch_shapes=[
            pltpu.VMEM((2, G, HIST * BB), jnp.float32),
            pltpu.VMEM((HIST, D, BB), jnp.float32),
            pltpu.VMEM((HIST, D, BB), jnp.float32),
            pltpu.VMEM((HIST, BB), jnp.float32),
        ],
    )(w_e, wd_c, bih_c, w_hh, bhh_c, g_u, g_i, m_u, m_i, du, di)


R_CH = 20000
NCH = V // R_CH


def _merge_body(ids_s, perm, bounds, mem, emb, out):
    c = pl.program_id(0)
    out[...] = mem[...]
    base = c * R_CH

    def patch(j, _):
        r = ids_s[j] - base
        srow = perm[j]
        out[pl.ds(r, 1), :] = emb[pl.ds(srow, 1), :]
        return 0

    lax.fori_loop(bounds[c], bounds[c + 1], patch, 0)


def _merge_copy(mem, emb, ids_s, perm, bounds):
    smem = pl.BlockSpec(memory_space=pltpu.SMEM)
    return pl.pallas_call(
        _merge_body,
        grid=(NCH,),
        in_specs=[
            smem, smem, smem,
            pl.BlockSpec((R_CH, D), lambda i: (i, 0)),
            pl.BlockSpec((B, D), lambda i: (0, 0)),
        ],
        out_specs=pl.BlockSpec((R_CH, D), lambda i: (i, 0)),
        out_shape=jax.ShapeDtypeStruct((V, D), jnp.float32),
    )(ids_s, perm, bounds, mem, emb)


def kernel(user_ids, user_features, item_ids, item_features,
           user_table, item_table, W_ih, W_hh, b_ih, b_hh,
           user_memory, item_memory):
    w_e = W_ih[:, :D]
    wd_c = W_ih[:, D].reshape(G, 1)
    bih_c = b_ih.reshape(G, 1)
    w_hh = W_hh
    bhh_c = b_hh.reshape(G, 1)
    def blockmajor(x):
        return x.T.reshape(HIST, NB, BB).transpose(1, 0, 2).reshape(GN)

    u_hist_ids = blockmajor(user_features[:, ::2].astype(jnp.int32))
    i_hist_ids = blockmajor(item_features[:, ::2].astype(jnp.int32))
    du = (user_features[:, 1::2].T.reshape(HIST, NB, BB)
          .transpose(1, 0, 2).reshape(NB, 1, HIST * BB))
    di = (item_features[:, 1::2].T.reshape(HIST, NB, BB)
          .transpose(1, 0, 2).reshape(NB, 1, HIST * BB))
    u_idx = u_hist_ids >> 1
    i_idx = i_hist_ids >> 1
    m_u = jnp.broadcast_to((u_hist_ids & 1).astype(jnp.float32)[:, None], (GN, D))
    m_i = jnp.broadcast_to((i_hist_ids & 1).astype(jnp.float32)[:, None], (GN, D))
    pt_item = item_table[1:].reshape(PV, 2 * D)
    pt_user = user_table[1:].reshape(PV, 2 * D)

    g_u, g_i = _make_sc_gather()(pt_item, pt_user, u_idx, i_idx)

    ue, ie = _tc_main(w_e, wd_c, bih_c, w_hh, bhh_c, g_u, g_i, m_u, m_i, du, di)

    grid_edges = jnp.arange(0, V + 1, R_CH, dtype=jnp.int32)
    outs = []
    for ids, emb, mem in ((user_ids, ue, user_memory), (item_ids, ie, item_memory)):
        ids = ids.astype(jnp.int32)
        perm = jnp.argsort(ids, stable=True).astype(jnp.int32)
        ids_s = ids[perm]
        bounds = jnp.searchsorted(ids_s, grid_edges).astype(jnp.int32)
        outs.append(_merge_copy(mem, emb, ids_s, perm, bounds))
    return (ue, ie, outs[0], outs[1])


# speedup vs baseline: 1.0124x; 1.0124x over previous
import jax
import jax.numpy as jnp
from jax import lax
from jax.experimental import pallas as pl
from jax.experimental.pallas import tpu as pltpu
from jax.experimental.pallas import tpu_sc as plsc

B = 4096
HIST = 20
D = 64
G = 3 * D
V = 1000000

BB = 256
NB = B // BB

NC, NS = 2, 16
NW = NC * NS
GN = B * HIST
G_PER_W = GN // NW
G_CH = G_PER_W // 4
S_PER_W = B // NW


def _sc_gather_body(item_t, user_t, u_idx, i_idx, g_u, g_i,
                    idx_v, rows_v, sem):
    wid = lax.axis_index("s") * NC + lax.axis_index("c")
    for table, idx_hbm, out_hbm in ((item_t, u_idx, g_u), (user_t, i_idx, g_i)):
        for c in range(G_PER_W // G_CH):
            base = wid * G_PER_W + c * G_CH
            pltpu.sync_copy(idx_hbm.at[pl.ds(base, G_CH)], idx_v)
            pltpu.async_copy(table.at[idx_v], rows_v, sem).wait()
            pltpu.sync_copy(rows_v, out_hbm.at[pl.ds(base, G_CH)])


PV = V // 2


import functools as _ft


@_ft.cache
def _make_sc_gather():
  return pl.kernel(
    _sc_gather_body,
    out_type=[jax.ShapeDtypeStruct((GN, 2 * D), jnp.float32)] * 2,
    mesh=plsc.VectorSubcoreMesh(core_axis_name="c", subcore_axis_name="s", num_cores=NC, num_subcores=NS),
    scratch_types=[
        pltpu.VMEM((G_CH,), jnp.int32),
        pltpu.VMEM((G_CH, 2 * D), jnp.float32),
        pltpu.SemaphoreType.DMA,
    ],
  )


def _tc_body(w_e, wd_c, bih_c, w_hh, bhh_c, gu, gi_, mu, mi, du, di,
             ue_out, ie_out,
             x_ref, hu_ref, hi_ref, ulog_ref):
    we = w_e[...]
    whh = w_hh[...]
    b_i = bih_c[...]
    b_h = bhh_c[...]
    wdv = wd_c[...]

    nt = (((1,), (1,)), ((), ()))
    for e, g, mk, dl in ((0, gu, mu, du), (1, gi_, mi, di)):
        e128 = g[...]
        lo = e128[:, :D]
        hi = e128[:, D:]
        m = mk[...].astype(jnp.float32)
        emb = lo + m * (hi - lo)
        giT = lax.dot_general(we, emb, nt, preferred_element_type=jnp.float32)
        giT = giT + wdv * dl[...].reshape(1, HIST * BB) + b_i
        x_ref[e] = giT

    nn = (((1,), (0,)), ((), ()))

    def step(t, HT):
        xu = x_ref[0, :, pl.ds(t * BB, BB)]
        xi = x_ref[1, :, pl.ds(t * BB, BB)]
        xt = jnp.concatenate([xu, xi], axis=1)
        gh = lax.dot_general(whh, HT, nn, preferred_element_type=jnp.float32) + b_h
        r = jax.nn.sigmoid(xt[:D] + gh[:D])
        z = jax.nn.sigmoid(xt[D:2 * D] + gh[D:2 * D])
        n = jnp.tanh(xt[2 * D:] + r * gh[2 * D:])
        HTn = (1.0 - z) * n + z * HT
        hu_ref[t] = HTn[:, :BB]
        hi_ref[t] = HTn[:, BB:]
        return HTn

    lax.fori_loop(0, HIST, step, jnp.zeros((D, 2 * BB), jnp.float32))

    def att_h(h, i_acc):
        u_h = hu_ref[h]
        ths = []
        for k in range(HIST):
            ths.append(jnp.tanh(jnp.sum(u_h * hi_ref[k], axis=0)))
        th = jnp.stack(ths, axis=0)
        ulog_ref[h] = jnp.mean(th, axis=0)
        return i_acc + th

    i_sum = lax.fori_loop(0, HIST, att_h, jnp.zeros((HIST, BB), jnp.float32))

    def soft(x):
        m = jnp.max(x, axis=0, keepdims=True)
        e = jnp.exp(x - m)
        return e / jnp.sum(e, axis=0, keepdims=True)

    u_att = soft(ulog_ref[...])
    i_att = soft(i_sum / HIST)
    ueT = jnp.zeros((D, BB), jnp.float32)
    ieT = jnp.zeros((D, BB), jnp.float32)
    for h in range(HIST):
        ueT = ueT + u_att[h][None, :] * hu_ref[h]
        ieT = ieT + i_att[h][None, :] * hi_ref[h]
    ue_out[...] = ueT.T
    ie_out[...] = ieT.T


def _tc_main(w_e, wd_c, bih_c, w_hh, bhh_c, g_u, g_i, m_u, m_i, du, di):
    full = lambda s: pl.BlockSpec(s, lambda i: (0,) * len(s))
    return pl.pallas_call(
        _tc_body,
        grid=(NB,),
        in_specs=[
            full((G, D)), full((G, 1)), full((G, 1)), full((G, D)), full((G, 1)),
            pl.BlockSpec((HIST * BB, 2 * D), lambda i: (i, 0)),
            pl.BlockSpec((HIST * BB, 2 * D), lambda i: (i, 0)),
            pl.BlockSpec((HIST * BB, D), lambda i: (i, 0)),
            pl.BlockSpec((HIST * BB, D), lambda i: (i, 0)),
            pl.BlockSpec((1, 1, HIST * BB), lambda i: (i, 0, 0)),
            pl.BlockSpec((1, 1, HIST * BB), lambda i: (i, 0, 0)),
        ],
        out_specs=[pl.BlockSpec((BB, D), lambda i: (i, 0))] * 2,
        out_shape=[jax.ShapeDtypeStruct((B, D), jnp.float32)] * 2,
        scratch_shapes=[
            pltpu.VMEM((2, G, HIST * BB), jnp.float32),
            pltpu.VMEM((HIST, D, BB), jnp.float32),
            pltpu.VMEM((HIST, D, BB), jnp.float32),
            pltpu.VMEM((HIST, BB), jnp.float32),
        ],
    )(w_e, wd_c, bih_c, w_hh, bhh_c, g_u, g_i, m_u, m_i, du, di)


R_CH = 20000
NCH = V // R_CH


def _merge_body(ids_s, perm, bounds, mem, emb, out):
    c = pl.program_id(0)
    out[...] = mem[...]
    base = c * R_CH

    def patch(j, _):
        r = ids_s[j] - base
        srow = perm[j]
        out[pl.ds(r, 1), :] = emb[pl.ds(srow, 1), :]
        return 0

    lax.fori_loop(bounds[c], bounds[c + 1], patch, 0)


def _merge_copy(mem, emb, ids_s, perm, bounds):
    smem = pl.BlockSpec(memory_space=pltpu.SMEM)
    return pl.pallas_call(
        _merge_body,
        grid=(NCH,),
        in_specs=[
            smem, smem, smem,
            pl.BlockSpec((R_CH, D), lambda i: (i, 0)),
            pl.BlockSpec((B, D), lambda i: (0, 0)),
        ],
        out_specs=pl.BlockSpec((R_CH, D), lambda i: (i, 0)),
        out_shape=jax.ShapeDtypeStruct((V, D), jnp.float32),
    )(ids_s, perm, bounds, mem, emb)


def kernel(user_ids, user_features, item_ids, item_features,
           user_table, item_table, W_ih, W_hh, b_ih, b_hh,
           user_memory, item_memory):
    w_e = W_ih[:, :D]
    wd_c = W_ih[:, D].reshape(G, 1)
    bih_c = b_ih.reshape(G, 1)
    w_hh = W_hh
    bhh_c = b_hh.reshape(G, 1)
    def blockmajor(x):
        return x.T.reshape(HIST, NB, BB).transpose(1, 0, 2).reshape(GN)

    u_hist_ids = blockmajor(user_features[:, ::2].astype(jnp.int32))
    i_hist_ids = blockmajor(item_features[:, ::2].astype(jnp.int32))
    du = (user_features[:, 1::2].T.reshape(HIST, NB, BB)
          .transpose(1, 0, 2).reshape(NB, 1, HIST * BB))
    di = (item_features[:, 1::2].T.reshape(HIST, NB, BB)
          .transpose(1, 0, 2).reshape(NB, 1, HIST * BB))
    u_idx = u_hist_ids >> 1
    i_idx = i_hist_ids >> 1
    m_u = jnp.broadcast_to((u_hist_ids & 1).astype(jnp.int8)[:, None], (GN, D))
    m_i = jnp.broadcast_to((i_hist_ids & 1).astype(jnp.int8)[:, None], (GN, D))
    pt_item = item_table[1:].reshape(PV, 2 * D)
    pt_user = user_table[1:].reshape(PV, 2 * D)

    g_u, g_i = _make_sc_gather()(pt_item, pt_user, u_idx, i_idx)

    ue, ie = _tc_main(w_e, wd_c, bih_c, w_hh, bhh_c, g_u, g_i, m_u, m_i, du, di)

    grid_edges = jnp.arange(0, V + 1, R_CH, dtype=jnp.int32)
    outs = []
    for ids, emb, mem in ((user_ids, ue, user_memory), (item_ids, ie, item_memory)):
        ids = ids.astype(jnp.int32)
        perm = jnp.argsort(ids, stable=True).astype(jnp.int32)
        ids_s = ids[perm]
        bounds = jnp.searchsorted(ids_s, grid_edges).astype(jnp.int32)
        outs.append(_merge_copy(mem, emb, ids_s, perm, bounds))
    return (ue, ie, outs[0], outs[1])


# speedup vs baseline: 1.7680x; 1.7465x over previous
import jax
import jax.numpy as jnp
from jax import lax
from jax.experimental import pallas as pl
from jax.experimental.pallas import tpu as pltpu
from jax.experimental.pallas import tpu_sc as plsc

B = 4096
HIST = 20
D = 64
G = 3 * D
V = 1000000

BB = 256
NB = B // BB

NC, NS = 2, 16
NW = NC * NS
GN = B * HIST
G_PER_W = GN // NW
G_CH = G_PER_W // 4
S_PER_W = B // NW


def _sc_gather_body(item_t, user_t, u_idx, i_idx, g_u, g_i,
                    idx_v, rows_v, sem):
    wid = lax.axis_index("s") * NC + lax.axis_index("c")
    for table, idx_hbm, out_hbm in ((item_t, u_idx, g_u), (user_t, i_idx, g_i)):
        for c in range(G_PER_W // (2 * G_CH)):
            base = wid * G_PER_W + c * 2 * G_CH
            pltpu.sync_copy(idx_hbm.at[pl.ds(base, 2 * G_CH)], idx_v)
            pltpu.async_copy(table.at[idx_v], rows_v, sem).wait()
            pltpu.sync_copy(rows_v, out_hbm.at[pl.ds(base, 2 * G_CH)])


PV = V // 2


import functools as _ft


@_ft.cache
def _make_sc_gather():
  return pl.kernel(
    _sc_gather_body,
    out_type=[jax.ShapeDtypeStruct((GN, D), jnp.float32)] * 2,
    mesh=plsc.VectorSubcoreMesh(core_axis_name="c", subcore_axis_name="s", num_cores=NC, num_subcores=NS),
    compiler_params=pltpu.CompilerParams(use_tc_tiling_on_sc=False),
    scratch_types=[
        pltpu.VMEM((2 * G_CH,), jnp.int32),
        pltpu.VMEM((2 * G_CH, D), jnp.float32),
        pltpu.SemaphoreType.DMA,
    ],
  )


def _tc_body(w_e, wd_c, bih_c, w_hh, bhh_c, gu, gi_, du, di,
             ue_out, ie_out,
             x_ref, hu_ref, hi_ref, ulog_ref):
    we = w_e[...]
    whh = w_hh[...]
    b_i = bih_c[...]
    b_h = bhh_c[...]
    wdv = wd_c[...]

    nt = (((1,), (1,)), ((), ()))
    for e, g, dl in ((0, gu, du), (1, gi_, di)):
        emb = g[...]
        giT = lax.dot_general(we, emb, nt, preferred_element_type=jnp.float32)
        giT = giT + wdv * dl[...].reshape(1, HIST * BB) + b_i
        x_ref[e] = giT

    nn = (((1,), (0,)), ((), ()))

    def step(t, HT):
        xu = x_ref[0, :, pl.ds(t * BB, BB)]
        xi = x_ref[1, :, pl.ds(t * BB, BB)]
        xt = jnp.concatenate([xu, xi], axis=1)
        gh = lax.dot_general(whh, HT, nn, preferred_element_type=jnp.float32) + b_h
        r = jax.nn.sigmoid(xt[:D] + gh[:D])
        z = jax.nn.sigmoid(xt[D:2 * D] + gh[D:2 * D])
        n = jnp.tanh(xt[2 * D:] + r * gh[2 * D:])
        HTn = (1.0 - z) * n + z * HT
        hu_ref[t] = HTn[:, :BB]
        hi_ref[t] = HTn[:, BB:]
        return HTn

    lax.fori_loop(0, HIST, step, jnp.zeros((D, 2 * BB), jnp.float32))

    def att_h(h, i_acc):
        u_h = hu_ref[h]
        ths = []
        for k in range(HIST):
            ths.append(jnp.tanh(jnp.sum(u_h * hi_ref[k], axis=0)))
        th = jnp.stack(ths, axis=0)
        ulog_ref[h] = jnp.mean(th, axis=0)
        return i_acc + th

    i_sum = lax.fori_loop(0, HIST, att_h, jnp.zeros((HIST, BB), jnp.float32))

    def soft(x):
        m = jnp.max(x, axis=0, keepdims=True)
        e = jnp.exp(x - m)
        return e / jnp.sum(e, axis=0, keepdims=True)

    u_att = soft(ulog_ref[...])
    i_att = soft(i_sum / HIST)
    ueT = jnp.zeros((D, BB), jnp.float32)
    ieT = jnp.zeros((D, BB), jnp.float32)
    for h in range(HIST):
        ueT = ueT + u_att[h][None, :] * hu_ref[h]
        ieT = ieT + i_att[h][None, :] * hi_ref[h]
    ue_out[...] = ueT
    ie_out[...] = ieT


def _tc_main(w_e, wd_c, bih_c, w_hh, bhh_c, g_u, g_i, du, di):
    full = lambda s: pl.BlockSpec(s, lambda i: (0,) * len(s))
    return pl.pallas_call(
        _tc_body,
        grid=(NB,),
        in_specs=[
            full((G, D)), full((G, 1)), full((G, 1)), full((G, D)), full((G, 1)),
            pl.BlockSpec((HIST * BB, D), lambda i: (i, 0)),
            pl.BlockSpec((HIST * BB, D), lambda i: (i, 0)),
            pl.BlockSpec((1, 1, HIST * BB), lambda i: (i, 0, 0)),
            pl.BlockSpec((1, 1, HIST * BB), lambda i: (i, 0, 0)),
        ],
        out_specs=[pl.BlockSpec((D, BB), lambda i: (0, i))] * 2,
        out_shape=[jax.ShapeDtypeStruct((D, B), jnp.float32)] * 2,
        scratch_shapes=[
            pltpu.VMEM((2, G, HIST * BB), jnp.float32),
            pltpu.VMEM((HIST, D, BB), jnp.float32),
            pltpu.VMEM((HIST, D, BB), jnp.float32),
            pltpu.VMEM((HIST, BB), jnp.float32),
        ],
    )(w_e, wd_c, bih_c, w_hh, bhh_c, g_u, g_i, du, di)


R_CH = 25600
NCH = -(-V // R_CH)


def _merge_body(ids_s, perm, bounds, memT, embT, out):
    c = pl.program_id(0)
    out[...] = memT[...]
    base = c * R_CH
    lane = lax.broadcasted_iota(jnp.int32, (1, 128), 1)

    def patch(j, _):
        col = ids_s[j] - base
        c0 = pl.multiple_of((col >> 7) << 7, 128)
        cm = col & 127
        src = perm[j]
        s0 = pl.multiple_of((src >> 7) << 7, 128)
        sm = src & 127
        swin = embT[:, pl.ds(s0, 128)]
        scol = jnp.sum(jnp.where(lane == sm, swin, 0.0), axis=1, keepdims=True)
        win = out[:, pl.ds(c0, 128)]
        out[:, pl.ds(c0, 128)] = jnp.where(lane == cm, scol, win)
        return 0

    lax.fori_loop(bounds[c], bounds[c + 1], patch, 0)


def _merge_copy(memT, embT, ids_s, perm, bounds):
    smem = pl.BlockSpec(memory_space=pltpu.SMEM)
    return pl.pallas_call(
        _merge_body,
        grid=(NCH,),
        in_specs=[
            smem, smem, smem,
            pl.BlockSpec((D, R_CH), lambda i: (0, i)),
            pl.BlockSpec((D, B), lambda i: (0, 0)),
        ],
        out_specs=pl.BlockSpec((D, R_CH), lambda i: (0, i)),
        out_shape=jax.ShapeDtypeStruct((D, V), jnp.float32),
    )(ids_s, perm, bounds, memT, embT)


def kernel(user_ids, user_features, item_ids, item_features,
           user_table, item_table, W_ih, W_hh, b_ih, b_hh,
           user_memory, item_memory):
    w_e = W_ih[:, :D]
    wd_c = W_ih[:, D].reshape(G, 1)
    bih_c = b_ih.reshape(G, 1)
    w_hh = W_hh
    bhh_c = b_hh.reshape(G, 1)
    def blockmajor(x):
        return x.T.reshape(HIST, NB, BB).transpose(1, 0, 2).reshape(GN)

    u_hist_ids = blockmajor(user_features[:, ::2].astype(jnp.int32)) + 1
    i_hist_ids = blockmajor(item_features[:, ::2].astype(jnp.int32)) + 1
    du = (user_features[:, 1::2].T.reshape(HIST, NB, BB)
          .transpose(1, 0, 2).reshape(NB, 1, HIST * BB))
    di = (item_features[:, 1::2].T.reshape(HIST, NB, BB)
          .transpose(1, 0, 2).reshape(NB, 1, HIST * BB))
    g_u, g_i = _make_sc_gather()(item_table, user_table, u_hist_ids, i_hist_ids)

    ueT, ieT = _tc_main(w_e, wd_c, bih_c, w_hh, bhh_c, g_u, g_i, du, di)

    grid_edges = jnp.arange(0, (NCH + 1) * R_CH, R_CH, dtype=jnp.int32)
    outs = []
    for ids, embT, mem in ((user_ids, ueT, user_memory), (item_ids, ieT, item_memory)):
        ids = ids.astype(jnp.int32)
        perm = jnp.argsort(ids, stable=True).astype(jnp.int32)
        ids_s = ids[perm]
        bounds = jnp.searchsorted(ids_s, grid_edges).astype(jnp.int32)
        outs.append(_merge_copy(mem.T, embT, ids_s, perm, bounds).T)
    return (ueT.T, ieT.T, outs[0], outs[1])
